# Initial kernel scaffold; baseline (speedup 1.0000x reference)
import jax
import jax.numpy as jnp
from jax.experimental import pallas as pl


def kernel(hidden, attention_mask, W1, b1, W2, b2):
    raise NotImplementedError("write your pallas kernel here")


# speedup vs baseline: 1.0380x; 1.0380x over previous
import functools

import numpy as np
import jax
import jax.numpy as jnp
from jax import lax
from jax.experimental import pallas as pl
from jax.experimental.pallas import tpu as pltpu

_BT = 1024

_INV_SQRT2 = 0.7071067811865476


def _body(x_ref, m_ref, w1_ref, b1_ref, w2_ref, b2_ref, tri_ref,
          logits_ref, probs_ref, head_ref, t2s_ref, carry_ref,
          *, blocks_per_batch):
    i = pl.program_id(0)

    @pl.when(i % blocks_per_batch == 0)
    def _():
        carry_ref[0, 0] = 0.0

    x = x_ref[...]
    h = jnp.dot(x, w1_ref[...], preferred_element_type=jnp.float32)
    h = h + b1_ref[...]
    h = 0.5 * h * (1.0 + lax.erf(h * _INV_SQRT2))
    logit = jnp.dot(h, w2_ref[...], preferred_element_type=jnp.float32)
    logit = logit + b2_ref[...]
    prob = jax.nn.sigmoid(logit)
    hard = (prob > 0.5).astype(jnp.float32)

    logits_ref[...] = logit
    probs_ref[...] = prob
    head_ref[...] = hard

    csum = jnp.dot(tri_ref[...], hard, preferred_element_type=jnp.float32)
    c0 = carry_ref[0, 0]
    m = m_ref[...]
    t2s = (csum + (c0 - 1.0)) * m - (1.0 - m)
    t2s_ref[...] = t2s.astype(jnp.int32)
    carry_ref[0, 0] = c0 + jnp.sum(hard)


def kernel(hidden, attention_mask, W1, b1, W2, b2):
    B, N, D = hidden.shape
    H = W1.shape[1]
    T = B * N
    bt = _BT
    nblk = T // bt
    blocks_per_batch = N // bt

    x = hidden.reshape(T, D)
    m = attention_mask.reshape(T, 1)
    b1r = b1.reshape(1, H)
    b2r = b2.reshape(1, 1)
    tri = jnp.asarray(np.tril(np.ones((bt, bt), np.float32)))

    out_shape = (
        jax.ShapeDtypeStruct((T, 1), jnp.float32),
        jax.ShapeDtypeStruct((T, 1), jnp.float32),
        jax.ShapeDtypeStruct((T, 1), jnp.float32),
        jax.ShapeDtypeStruct((T, 1), jnp.int32),
    )

    tok_spec = pl.BlockSpec((bt, 1), lambda i: (i, 0))
    const = lambda i: (0, 0)

    logits, probs, head, t2s = pl.pallas_call(
        functools.partial(_body, blocks_per_batch=blocks_per_batch),
        grid=(nblk,),
        in_specs=[
            pl.BlockSpec((bt, D), lambda i: (i, 0)),
            tok_spec,
            pl.BlockSpec((D, H), const),
            pl.BlockSpec((1, H), const),
            pl.BlockSpec((H, 1), const),
            pl.BlockSpec((1, 1), const),
            pl.BlockSpec((bt, bt), const),
        ],
        out_specs=[tok_spec, tok_spec, tok_spec, tok_spec],
        out_shape=out_shape,
        scratch_shapes=[pltpu.SMEM((1, 1), jnp.float32)],
        compiler_params=pltpu.CompilerParams(
            dimension_semantics=("arbitrary",),
        ),
    )(x, m, W1, b1r, W2, b2r, tri)

    return (
        logits.reshape(B, N),
        probs.reshape(B, N),
        head.reshape(B, N),
        t2s.reshape(B, N),
    )
